# baseline (device time: 19358 ns/iter reference)
import jax
import jax.numpy as jnp
from jax import lax
from jax.experimental import pallas as pl
from jax.experimental.pallas import tpu as pltpu


def kernel(partial, resid, gamma):
    _, m, d = partial.shape
    gamma2d = gamma.reshape(1, d)

    def body(partial_ref, resid_ref, gamma_ref, out_ref, comm_ref,
             send_sem, recv_sem):
        my_x = lax.axis_index("x")
        my_y = lax.axis_index("y")
        my_z = lax.axis_index("z")
        other = (1 - my_x, my_y, my_z)

        barrier_sem = pltpu.get_barrier_semaphore()
        pl.semaphore_signal(
            barrier_sem, inc=1,
            device_id=other, device_id_type=pl.DeviceIdType.MESH,
        )
        pl.semaphore_wait(barrier_sem, 1)

        rdma = pltpu.make_async_remote_copy(
            src_ref=partial_ref,
            dst_ref=comm_ref,
            send_sem=send_sem,
            recv_sem=recv_sem,
            device_id=other,
            device_id_type=pl.DeviceIdType.MESH,
        )
        rdma.start()
        rdma.wait()

        y = partial_ref[0, :, :] + comm_ref[0, :, :] + resid_ref[:, :]
        ms = jnp.mean(y * y, axis=-1, keepdims=True)
        out_ref[:, :] = y * lax.rsqrt(ms + 1e-6) * gamma_ref[0, :]

    return pl.pallas_call(
        body,
        out_shape=jax.ShapeDtypeStruct((m, d), jnp.float32),
        in_specs=[
            pl.BlockSpec(memory_space=pltpu.VMEM),
            pl.BlockSpec(memory_space=pltpu.VMEM),
            pl.BlockSpec(memory_space=pltpu.VMEM),
        ],
        out_specs=pl.BlockSpec(memory_space=pltpu.VMEM),
        scratch_shapes=[
            pltpu.VMEM((1, m, d), jnp.float32),
            pltpu.SemaphoreType.DMA,
            pltpu.SemaphoreType.DMA,
        ],
        compiler_params=pltpu.CompilerParams(collective_id=0),
    )(partial, resid, gamma2d)


# device time: 16956 ns/iter; 1.1417x vs baseline; 1.1417x over previous
import jax
import jax.numpy as jnp
from jax import lax
from jax.experimental import pallas as pl
from jax.experimental.pallas import tpu as pltpu

K = 8


def kernel(partial, resid, gamma):
    _, m, d = partial.shape
    half = m // 2
    rpc = half // K
    gamma2d = gamma.reshape(1, d)

    def body(partial_ref, resid_ref, gamma_ref, out_ref, xbuf,
             xsend_sems, xrecv_sems, ysend_sems, yrecv_sems):
        my_x = lax.axis_index("x")
        my_y = lax.axis_index("y")
        my_z = lax.axis_index("z")
        xpeer = (1 - my_x, my_y, my_z)
        ypeer = (my_x, my_y ^ 1, my_z)

        h = my_y % 2
        base = h * half

        barrier_sem = pltpu.get_barrier_semaphore()
        for peer in (xpeer, ypeer):
            pl.semaphore_signal(
                barrier_sem, inc=1,
                device_id=peer, device_id_type=pl.DeviceIdType.MESH,
            )
        pl.semaphore_wait(barrier_sem, 2)

        xrdmas = []
        for k in range(K):
            rows = pl.ds(base + k * rpc, rpc)
            rdma = pltpu.make_async_remote_copy(
                src_ref=partial_ref.at[0, rows, :],
                dst_ref=xbuf.at[k],
                send_sem=xsend_sems.at[k],
                recv_sem=xrecv_sems.at[k],
                device_id=xpeer,
                device_id_type=pl.DeviceIdType.MESH,
            )
            rdma.start()
            xrdmas.append(rdma)

        yrdmas = []
        for k in range(K):
            xrdmas[k].wait_recv()
            rows = pl.ds(base + k * rpc, rpc)
            y = partial_ref[0, rows, :] + xbuf[k, :, :] + resid_ref[rows, :]
            ms = jnp.mean(y * y, axis=-1, keepdims=True)
            out_ref[rows, :] = y * lax.rsqrt(ms + 1e-6) * gamma_ref[0, :]
            yr = pltpu.make_async_remote_copy(
                src_ref=out_ref.at[rows, :],
                dst_ref=out_ref.at[rows, :],
                send_sem=ysend_sems.at[k],
                recv_sem=yrecv_sems.at[k],
                device_id=ypeer,
                device_id_type=pl.DeviceIdType.MESH,
            )
            yr.start()
            yrdmas.append(yr)

        for k in range(K):
            yrdmas[k].wait_recv()
        for k in range(K):
            xrdmas[k].wait_send()
            yrdmas[k].wait_send()

    return pl.pallas_call(
        body,
        out_shape=jax.ShapeDtypeStruct((m, d), jnp.float32),
        in_specs=[
            pl.BlockSpec(memory_space=pltpu.VMEM),
            pl.BlockSpec(memory_space=pltpu.VMEM),
            pl.BlockSpec(memory_space=pltpu.VMEM),
        ],
        out_specs=pl.BlockSpec(memory_space=pltpu.VMEM),
        scratch_shapes=[
            pltpu.VMEM((K, rpc, d), jnp.float32),
            pltpu.SemaphoreType.DMA((K,)),
            pltpu.SemaphoreType.DMA((K,)),
            pltpu.SemaphoreType.DMA((K,)),
            pltpu.SemaphoreType.DMA((K,)),
        ],
        compiler_params=pltpu.CompilerParams(collective_id=0),
    )(partial, resid, gamma2d)
